# Initial kernel scaffold; baseline (speedup 1.0000x reference)
import jax
import jax.numpy as jnp
from jax.experimental import pallas as pl


def kernel(x, edge_index, W1, b1, W2, b2):
    raise NotImplementedError("write your pallas kernel here")


# speedup vs baseline: 6.4853x; 6.4853x over previous
import jax
import jax.numpy as jnp
from jax import lax
from jax.experimental import pallas as pl
from jax.experimental.pallas import tpu as pltpu
from jax.experimental.pallas import tpu_sc as plsc

_N = 10000
_E = 320000
_D = 64
_DW = 128
_NF = 128
_NH = 128
_K = 10
_ALPHA = 0.1

_NCORES = 2
_NSUB = 16
_NW = _NCORES * _NSUB
_CH = 128
_CPW = 79
_EW = _CH * _CPW
_EPAD = _EW * _NW
_NPAD = 10112
_RS = _NPAD // _NSUB

_mesh = plsc.VectorSubcoreMesh(core_axis_name="c", subcore_axis_name="s")


def _scatter_body(u_hbm, srcp_hbm, dstp_hbm, zeros_hbm, out_hbm,
                  acc, sidx, didx, rows, sem):
    c = lax.axis_index("c")
    s = lax.axis_index("s")
    w = s * _NCORES + c
    pltpu.sync_copy(zeros_hbm.at[pl.ds(s * _RS, _RS)],
                    acc.at[pl.ds(s * _RS, _RS)])
    plsc.subcore_barrier()

    def chunk(j, carry):
        base = w * _EW + j * _CH
        pltpu.sync_copy(srcp_hbm.at[pl.ds(base, _CH)], sidx)
        pltpu.sync_copy(dstp_hbm.at[pl.ds(base, _CH)], didx)
        pltpu.async_copy(u_hbm.at[sidx], rows, sem).wait()
        pltpu.sync_copy(rows, acc.at[didx], add=True)
        return carry

    lax.fori_loop(0, _CPW, chunk, 0)
    plsc.subcore_barrier()
    pltpu.sync_copy(acc.at[pl.ds(s * _RS, _RS)],
                    out_hbm.at[c, pl.ds(s * _RS, _RS)])


_scatter = pl.kernel(
    _scatter_body,
    out_type=jax.ShapeDtypeStruct((_NCORES, _NPAD, _DW), jnp.float32),
    mesh=_mesh,
    scratch_types=[
        pltpu.VMEM_SHARED((_NPAD, _DW), jnp.float32),
        pltpu.VMEM((_CH,), jnp.int32),
        pltpu.VMEM((_CH,), jnp.int32),
        pltpu.VMEM((_CH, _DW), jnp.float32),
        pltpu.SemaphoreType.DMA,
    ],
)


def _deg_body(ones_hbm, dstp_hbm, zeros_hbm, out_hbm, acc, didx, rows, sem):
    c = lax.axis_index("c")
    s = lax.axis_index("s")
    w = s * _NCORES + c
    pltpu.sync_copy(zeros_hbm.at[pl.ds(s * _RS, _RS)],
                    acc.at[pl.ds(s * _RS, _RS)])
    pltpu.async_copy(ones_hbm, rows, sem).wait()
    plsc.subcore_barrier()

    def chunk(j, carry):
        base = w * _EW + j * _CH
        pltpu.sync_copy(dstp_hbm.at[pl.ds(base, _CH)], didx)
        pltpu.sync_copy(rows, acc.at[didx], add=True)
        return carry

    lax.fori_loop(0, _CPW, chunk, 0)
    plsc.subcore_barrier()
    pltpu.sync_copy(acc.at[pl.ds(s * _RS, _RS)],
                    out_hbm.at[c, pl.ds(s * _RS, _RS)])


_deg = pl.kernel(
    _deg_body,
    out_type=jax.ShapeDtypeStruct((_NCORES, _NPAD, _DW), jnp.float32),
    mesh=_mesh,
    scratch_types=[
        pltpu.VMEM_SHARED((_NPAD, _DW), jnp.float32),
        pltpu.VMEM((_CH,), jnp.int32),
        pltpu.VMEM((_CH, _DW), jnp.float32),
        pltpu.SemaphoreType.DMA,
    ],
)


_BM = 1264


def _mlp_body(x_ref, w1_ref, b1_ref, w2_ref, b2_ref, dp0_ref, dp1_ref,
              u0_ref, dinv_ref, dinv2_ref):
    h = jnp.dot(x_ref[...], w1_ref[...], preferred_element_type=jnp.float32)
    h = jnp.maximum(h + b1_ref[...], 0.0)
    h = jnp.dot(h, w2_ref[...], preferred_element_type=jnp.float32) + b2_ref[...]
    deg = dp0_ref[0][:, 0:1] + dp1_ref[0][:, 0:1] + 1.0
    dinv = lax.rsqrt(deg)
    u0_ref[...] = jnp.concatenate(
        [h * dinv, jnp.zeros((_BM, _DW - _D), jnp.float32)], axis=1)
    dinv_ref[...] = dinv
    dinv2_ref[...] = dinv * dinv


def _mlp(xp, w1, b1, w2, b2, dpart):
    grid = _NPAD // _BM
    return pl.pallas_call(
        _mlp_body,
        grid=(grid,),
        in_specs=[
            pl.BlockSpec((_BM, _NF), lambda i: (i, 0)),
            pl.BlockSpec((_NF, _NH), lambda i: (0, 0)),
            pl.BlockSpec((1, _NH), lambda i: (0, 0)),
            pl.BlockSpec((_NH, _D), lambda i: (0, 0)),
            pl.BlockSpec((1, _D), lambda i: (0, 0)),
            pl.BlockSpec((1, _BM, _DW), lambda i: (0, i, 0)),
            pl.BlockSpec((1, _BM, _DW), lambda i: (1, i, 0)),
        ],
        out_specs=[
            pl.BlockSpec((_BM, _DW), lambda i: (i, 0)),
            pl.BlockSpec((_BM, 1), lambda i: (i, 0)),
            pl.BlockSpec((_BM, 1), lambda i: (i, 0)),
        ],
        out_shape=[
            jax.ShapeDtypeStruct((_NPAD, _DW), jnp.float32),
            jax.ShapeDtypeStruct((_NPAD, 1), jnp.float32),
            jax.ShapeDtypeStruct((_NPAD, 1), jnp.float32),
        ],
    )(xp, w1, b1.reshape(1, _NH), w2, b2.reshape(1, _D), dpart, dpart)


def _upd_body(t0_ref, t1_ref, u_ref, u0_ref, dinv2_ref, out_ref):
    t = t0_ref[0] + t1_ref[0] + u_ref[...]
    out_ref[...] = ((1.0 - _ALPHA) * dinv2_ref[...] * t
                    + _ALPHA * u0_ref[...])


def _update(tpart, u, u0, dinv2):
    grid = _NPAD // _BM
    return pl.pallas_call(
        _upd_body,
        grid=(grid,),
        in_specs=[
            pl.BlockSpec((1, _BM, _DW), lambda i: (0, i, 0)),
            pl.BlockSpec((1, _BM, _DW), lambda i: (1, i, 0)),
            pl.BlockSpec((_BM, _DW), lambda i: (i, 0)),
            pl.BlockSpec((_BM, _DW), lambda i: (i, 0)),
            pl.BlockSpec((_BM, 1), lambda i: (i, 0)),
        ],
        out_specs=pl.BlockSpec((_BM, _DW), lambda i: (i, 0)),
        out_shape=jax.ShapeDtypeStruct((_NPAD, _DW), jnp.float32),
    )(tpart, tpart, u, u0, dinv2)


_BF = 2000


def _final_body(u_ref, dinv_ref, out_ref):
    h = u_ref[:, 0:_D] / dinv_ref[...]
    m = jnp.max(h, axis=1, keepdims=True)
    ls = m + jnp.log(jnp.sum(jnp.exp(h - m), axis=1, keepdims=True))
    out_ref[...] = h - ls


def _final(u, dinv):
    return pl.pallas_call(
        _final_body,
        grid=(_N // _BF,),
        in_specs=[
            pl.BlockSpec((_BF, _DW), lambda i: (i, 0)),
            pl.BlockSpec((_BF, 1), lambda i: (i, 0)),
        ],
        out_specs=pl.BlockSpec((_BF, _D), lambda i: (i, 0)),
        out_shape=jax.ShapeDtypeStruct((_N, _D), jnp.float32),
    )(u, dinv)


def kernel(x, edge_index, W1, b1, W2, b2):
    src = edge_index[0]
    dst = edge_index[1]
    pad = _EPAD - _E
    srcp = jnp.concatenate([src, jnp.zeros((pad,), jnp.int32)])
    dstp = jnp.concatenate([dst, jnp.full((pad,), _N, jnp.int32)])
    zeros = jnp.zeros((_NPAD, _DW), jnp.float32)
    ones = jnp.ones((_CH, _DW), jnp.float32)
    xp = jnp.concatenate([x, jnp.zeros((_NPAD - _N, _NF), jnp.float32)])

    dpart = _deg(ones, dstp, zeros)
    u0, dinv, dinv2 = _mlp(xp, W1, b1, W2, b2, dpart)

    u = u0
    for _ in range(_K):
        tpart = _scatter(u, srcp, dstp, zeros)
        u = _update(tpart, u, u0, dinv2)

    return _final(u, dinv)
